# Initial kernel scaffold; baseline (speedup 1.0000x reference)
import jax
import jax.numpy as jnp
from jax.experimental import pallas as pl


def kernel(idx, targets, token_embedding):
    raise NotImplementedError("write your pallas kernel here")


# speedup vs baseline: 1.3722x; 1.3722x over previous
import functools

import jax
import jax.numpy as jnp
from jax import lax
from jax.experimental import pallas as pl
from jax.experimental.pallas import tpu as pltpu
from jax.experimental.pallas import tpu_sc as plsc

V = 1000
C = 1000
N = 1024 * 200
NC, NS, L = 2, 16, 16
NW = NC * NS
BPW = N // NW
K = 64
NCHUNK = BPW // K


def _lse_body(table_ref, out_ref):
    x = table_ref[...]
    m = jnp.max(x, axis=1, keepdims=True)
    s = jnp.sum(jnp.exp(x - m), axis=1, keepdims=True)
    out_ref[...] = m + jnp.log(s)


def _lse_call(table):
    return pl.pallas_call(
        _lse_body,
        out_shape=jax.ShapeDtypeStruct((V, 1), jnp.float32),
    )(table)


def _sc_body(table_hbm, idx_hbm, tgt_hbm, lse_hbm,
             logits_hbm, part_hbm,
             idx_v, tgt_v, lse_v, rows_v, acc_v, sem):
    wid = lax.axis_index("s") * NC + lax.axis_index("c")
    base = wid * BPW
    pltpu.sync_copy(idx_hbm.at[pl.ds(base, BPW)], idx_v)
    pltpu.sync_copy(tgt_hbm.at[pl.ds(base, BPW)], tgt_v)
    pltpu.sync_copy(lse_hbm, lse_v)

    lane = lax.iota(jnp.int32, L)

    def chunk_body(ci, acc):
        off = ci * K
        pltpu.async_copy(table_hbm.at[idx_v.at[pl.ds(off, K)]], rows_v,
                         sem).wait()
        pltpu.sync_copy(rows_v, logits_hbm.at[pl.ds(base + off, K)])

        def grp(g, a):
            o2 = off + g * L
            idx16 = idx_v[pl.ds(o2, L)]
            tgt16 = tgt_v[pl.ds(o2, L)]
            rid = lane + g * L
            tlogit = plsc.load_gather(rows_v, [rid, tgt16])
            lse16 = plsc.load_gather(lse_v, [idx16])
            return a + lse16 - tlogit

        return lax.fori_loop(0, K // L, grp, acc)

    acc = lax.fori_loop(0, NCHUNK, chunk_body, jnp.zeros((L,), jnp.float32))
    acc_v[...] = acc
    pltpu.sync_copy(acc_v, part_hbm.at[wid])


def _sc_call(table, idx_flat, tgt_flat, lse_flat):
    mesh = plsc.VectorSubcoreMesh(core_axis_name="c", subcore_axis_name="s")
    kern = functools.partial(
        pl.kernel,
        out_type=(jax.ShapeDtypeStruct((N, C), jnp.float32),
                  jax.ShapeDtypeStruct((NW, L), jnp.float32)),
        mesh=mesh,
        scratch_types=[
            pltpu.VMEM((BPW,), jnp.int32),
            pltpu.VMEM((BPW,), jnp.int32),
            pltpu.VMEM((V,), jnp.float32),
            pltpu.VMEM((K, C), jnp.float32),
            pltpu.VMEM((L,), jnp.float32),
            pltpu.SemaphoreType.DMA,
        ],
        compiler_params=pltpu.CompilerParams(use_tc_tiling_on_sc=False,
                                             needs_layout_passes=False),
    )(_sc_body)
    return kern(table, idx_flat, tgt_flat, lse_flat)


def _loss_body(part_ref, out_ref):
    out_ref[0, 0] = jnp.sum(part_ref[...]) * (1.0 / N)


def _loss_call(partials):
    return pl.pallas_call(
        _loss_body,
        out_shape=jax.ShapeDtypeStruct((1, 1), jnp.float32),
        out_specs=pl.BlockSpec(memory_space=pltpu.SMEM),
    )(partials)


def kernel(idx, targets, token_embedding):
    idx_flat = idx.reshape(-1).astype(jnp.int32)
    tgt_flat = targets.reshape(-1).astype(jnp.int32)
    lse = _lse_call(token_embedding).reshape(-1)
    logits, partials = _sc_call(token_embedding, idx_flat, tgt_flat, lse)
    loss = _loss_call(partials).reshape(())
    return (logits, loss)


# speedup vs baseline: 1.4025x; 1.0221x over previous
import functools

import jax
import jax.numpy as jnp
from jax import lax
from jax.experimental import pallas as pl
from jax.experimental.pallas import tpu as pltpu
from jax.experimental.pallas import tpu_sc as plsc

V = 1000
C = 1000
N = 1024 * 200
NC, NS, L = 2, 16, 16
NW = NC * NS
BPW = N // NW
K = 32
NCHUNK = BPW // K
NBUF = 2


def _lse_body(table_ref, out_ref):
    x = table_ref[...]
    m = jnp.max(x, axis=1, keepdims=True)
    s = jnp.sum(jnp.exp(x - m), axis=1, keepdims=True)
    out_ref[...] = m + jnp.log(s)


def _lse_call(table):
    return pl.pallas_call(
        _lse_body,
        out_shape=jax.ShapeDtypeStruct((V, 1), jnp.float32),
    )(table)


def _sc_body(table_hbm, idx_hbm, tgt_hbm, lse_hbm,
             logits_hbm, part_hbm,
             idx_v, tgt_v, lse_v, rows0, rows1, acc_v,
             gsem0, gsem1, ssem0, ssem1):
    rows = (rows0, rows1)
    gsem = (gsem0, gsem1)
    ssem = (ssem0, ssem1)
    wid = lax.axis_index("s") * NC + lax.axis_index("c")
    base = wid * BPW
    pltpu.sync_copy(idx_hbm.at[pl.ds(base, BPW)], idx_v)
    pltpu.sync_copy(tgt_hbm.at[pl.ds(base, BPW)], tgt_v)
    pltpu.sync_copy(lse_hbm, lse_v)

    lane = lax.iota(jnp.int32, L)

    def gather_start(ci, b):
        pltpu.make_async_copy(
            table_hbm.at[idx_v.at[pl.ds(ci * K, K)]], rows[b], gsem[b]
        ).start()

    for b in range(NBUF):
        gather_start(b, b)

    def group_body(g, acc):
        accs = [acc]
        for b in range(NBUF):
            ci = g * NBUF + b
            off = ci * K
            pltpu.make_async_copy(
                table_hbm.at[idx_v.at[pl.ds(0, K)]], rows[b], gsem[b]
            ).wait()
            pltpu.make_async_copy(
                rows[b], logits_hbm.at[pl.ds(base + off, K)], ssem[b]
            ).start()

            def grp(j, a, _b=b, _off=off):
                o2 = _off + j * L
                idx16 = idx_v[pl.ds(o2, L)]
                tgt16 = tgt_v[pl.ds(o2, L)]
                rid = lane + j * L
                tlogit = plsc.load_gather(rows[_b], [rid, tgt16])
                lse16 = plsc.load_gather(lse_v, [idx16])
                return a + lse16 - tlogit

            a = accs[-1]
            for j in range(K // L):
                a = grp(j, a)
            accs.append(a)

            pltpu.make_async_copy(
                rows[b], logits_hbm.at[pl.ds(base, K)], ssem[b]
            ).wait()

            @pl.when(ci + NBUF < NCHUNK)
            def _():
                gather_start(ci + NBUF, b)

        return accs[-1]

    acc = lax.fori_loop(0, NCHUNK // NBUF, group_body,
                        jnp.zeros((L,), jnp.float32))
    acc_v[...] = acc
    pltpu.sync_copy(acc_v, part_hbm.at[wid])


def _sc_call(table, idx_flat, tgt_flat, lse_flat):
    mesh = plsc.VectorSubcoreMesh(core_axis_name="c", subcore_axis_name="s")
    kern = functools.partial(
        pl.kernel,
        out_type=(jax.ShapeDtypeStruct((N, C), jnp.float32),
                  jax.ShapeDtypeStruct((NW, L), jnp.float32)),
        mesh=mesh,
        scratch_types=[
            pltpu.VMEM((BPW,), jnp.int32),
            pltpu.VMEM((BPW,), jnp.int32),
            pltpu.VMEM((V,), jnp.float32),
            pltpu.VMEM((K, C), jnp.float32),
            pltpu.VMEM((K, C), jnp.float32),
            pltpu.VMEM((L,), jnp.float32),
            pltpu.SemaphoreType.DMA,
            pltpu.SemaphoreType.DMA,
            pltpu.SemaphoreType.DMA,
            pltpu.SemaphoreType.DMA,
        ],
        compiler_params=pltpu.CompilerParams(use_tc_tiling_on_sc=False,
                                             needs_layout_passes=False),
    )(_sc_body)
    return kern(table, idx_flat, tgt_flat, lse_flat)


def _loss_body(part_ref, out_ref):
    out_ref[0, 0] = jnp.sum(part_ref[...]) * (1.0 / N)


def _loss_call(partials):
    return pl.pallas_call(
        _loss_body,
        out_shape=jax.ShapeDtypeStruct((1, 1), jnp.float32),
        out_specs=pl.BlockSpec(memory_space=pltpu.SMEM),
    )(partials)


def kernel(idx, targets, token_embedding):
    idx_flat = idx.reshape(-1).astype(jnp.int32)
    tgt_flat = targets.reshape(-1).astype(jnp.int32)
    lse = _lse_call(token_embedding).reshape(-1)
    logits, partials = _sc_call(token_embedding, idx_flat, tgt_flat, lse)
    loss = _loss_call(partials).reshape(())
    return (logits, loss)


# speedup vs baseline: 2.3744x; 1.6930x over previous
import functools

import jax
import jax.numpy as jnp
from jax import lax
from jax.experimental import pallas as pl
from jax.experimental.pallas import tpu as pltpu
from jax.experimental.pallas import tpu_sc as plsc

V = 1000
C = 1000
CP = 1024
N = 1024 * 200
NC, NS, L = 2, 16, 16
NW = NC * NS
BPW = N // NW
K = 32
NCHUNK = BPW // K
NBUF = 2


def _lse_body(table_ref, out_ref):
    x = table_ref[...]
    m = jnp.max(x, axis=1, keepdims=True)
    s = jnp.sum(jnp.exp(x - m), axis=1, keepdims=True)
    out_ref[...] = m + jnp.log(s)


def _lse_call(table):
    return pl.pallas_call(
        _lse_body,
        out_shape=jax.ShapeDtypeStruct((V, 1), jnp.float32),
    )(table)


def _gather_body(table_hbm, idx_hbm, logits_hbm,
                 idx_v, rows0, rows1, gsem0, gsem1, ssem0, ssem1):
    rows = (rows0, rows1)
    gsem = (gsem0, gsem1)
    ssem = (ssem0, ssem1)
    wid = lax.axis_index("s") * NC + lax.axis_index("c")
    base = wid * BPW
    pltpu.sync_copy(idx_hbm.at[pl.ds(base, BPW)], idx_v)

    def gather_start(ci, b):
        pltpu.make_async_copy(
            table_hbm.at[idx_v.at[pl.ds(ci * K, K)]], rows[b], gsem[b]
        ).start()

    for b in range(NBUF):
        gather_start(b, b)

    def group_body(g, carry):
        for b in range(NBUF):
            ci = g * NBUF + b
            off = ci * K
            pltpu.make_async_copy(
                table_hbm.at[idx_v.at[pl.ds(0, K)]], rows[b], gsem[b]
            ).wait()
            pltpu.make_async_copy(
                rows[b], logits_hbm.at[pl.ds(base + off, K)], ssem[b]
            ).start()
            pltpu.make_async_copy(
                rows[b], logits_hbm.at[pl.ds(base, K)], ssem[b]
            ).wait()

            @pl.when(ci + NBUF < NCHUNK)
            def _():
                gather_start(ci + NBUF, b)

        return carry

    lax.fori_loop(0, NCHUNK // NBUF, group_body, 0)


def _gather_call(table, idx_flat):
    mesh = plsc.VectorSubcoreMesh(core_axis_name="c", subcore_axis_name="s")
    kern = functools.partial(
        pl.kernel,
        out_type=jax.ShapeDtypeStruct((N, CP), jnp.float32),
        mesh=mesh,
        scratch_types=[
            pltpu.VMEM((BPW,), jnp.int32),
            pltpu.VMEM((K, CP), jnp.float32),
            pltpu.VMEM((K, CP), jnp.float32),
            pltpu.SemaphoreType.DMA,
            pltpu.SemaphoreType.DMA,
            pltpu.SemaphoreType.DMA,
            pltpu.SemaphoreType.DMA,
        ],
        compiler_params=pltpu.CompilerParams(use_tc_tiling_on_sc=True),
    )(_gather_body)
    return kern(table, idx_flat)


def _nll_body(tflat_hbm, idx_hbm, tgt_hbm, lse_hbm, part_hbm,
              idx_v, tgt_v, fidx_v, tl_v, lse_v, acc_v, sem):
    wid = lax.axis_index("s") * NC + lax.axis_index("c")
    base = wid * BPW
    pltpu.sync_copy(idx_hbm.at[pl.ds(base, BPW)], idx_v)
    pltpu.sync_copy(tgt_hbm.at[pl.ds(base, BPW)], tgt_v)
    pltpu.sync_copy(lse_hbm, lse_v)

    def fidx_body(g, carry):
        o = g * L
        fidx_v[pl.ds(o, L)] = idx_v[pl.ds(o, L)] * C + tgt_v[pl.ds(o, L)]
        return carry

    lax.fori_loop(0, BPW // L, fidx_body, 0)

    pltpu.async_copy(tflat_hbm.at[fidx_v], tl_v, sem).wait()

    def acc_body(g, acc):
        o = g * L
        lse16 = plsc.load_gather(lse_v, [idx_v[pl.ds(o, L)]])
        return acc + lse16 - tl_v[pl.ds(o, L)]

    acc = lax.fori_loop(0, BPW // L, acc_body, jnp.zeros((L,), jnp.float32))
    acc_v[...] = acc
    pltpu.sync_copy(acc_v, part_hbm.at[wid])


def _nll_call(table_flat, idx_flat, tgt_flat, lse_flat):
    mesh = plsc.VectorSubcoreMesh(core_axis_name="c", subcore_axis_name="s")
    kern = functools.partial(
        pl.kernel,
        out_type=jax.ShapeDtypeStruct((NW, L), jnp.float32),
        mesh=mesh,
        scratch_types=[
            pltpu.VMEM((BPW,), jnp.int32),
            pltpu.VMEM((BPW,), jnp.int32),
            pltpu.VMEM((BPW,), jnp.int32),
            pltpu.VMEM((BPW,), jnp.float32),
            pltpu.VMEM((V,), jnp.float32),
            pltpu.VMEM((L,), jnp.float32),
            pltpu.SemaphoreType.DMA,
        ],
        compiler_params=pltpu.CompilerParams(use_tc_tiling_on_sc=False,
                                             needs_layout_passes=False),
    )(_nll_body)
    return kern(table_flat, idx_flat, tgt_flat, lse_flat)


def _loss_body(part_ref, out_ref):
    out_ref[0, 0] = jnp.sum(part_ref[...]) * (1.0 / N)


def _loss_call(partials):
    return pl.pallas_call(
        _loss_body,
        out_shape=jax.ShapeDtypeStruct((1, 1), jnp.float32),
        out_specs=pl.BlockSpec(memory_space=pltpu.SMEM),
    )(partials)


def kernel(idx, targets, token_embedding):
    idx_flat = idx.reshape(-1).astype(jnp.int32)
    tgt_flat = targets.reshape(-1).astype(jnp.int32)
    lse = _lse_call(token_embedding).reshape(-1)
    table_pad = jnp.pad(token_embedding, ((0, 0), (0, CP - C)))
    logits = _gather_call(table_pad, idx_flat)[:, :C]
    partials = _nll_call(token_embedding.reshape(-1), idx_flat, tgt_flat, lse)
    loss = _loss_call(partials).reshape(())
    return (logits, loss)
